# Initial kernel scaffold; baseline (speedup 1.0000x reference)
import jax
import jax.numpy as jnp
from jax.experimental import pallas as pl


def kernel(node_attr, edge_index, edge_attr, edge_sh, global_graph_embedding, ptr, W1, b1, W2, b2, bn_w, bn_b):
    raise NotImplementedError("write your pallas kernel here")


# speedup vs baseline: 1.2956x; 1.2956x over previous
import functools

import jax
import jax.numpy as jnp
import numpy as np
from jax import lax
from jax.experimental import pallas as pl
from jax.experimental.pallas import tpu as pltpu
from jax.experimental.pallas import tpu_sc as plsc

N = 10000
E = 160000
C = 16
D = 16
NEF = 16
HID = 16
ALPHA = 1.0 / np.sqrt(C * 1)

NC = 2
NS = 16
NW = NC * NS
EPW = E // NW
CHUNK = 125
NCH = EPW // CHUNK
ROWS_PER_TILE = N // NS

def _gather_rows_body(table_hbm, idx_hbm, out_hbm, idx_v, rows_v, sem):
    cid = lax.axis_index("c")
    sid = lax.axis_index("s")
    wid = sid * NC + cid
    pltpu.sync_copy(idx_hbm.at[pl.ds(wid * NCH, NCH)], idx_v)

    def body(j, carry):
        pltpu.async_copy(
            table_hbm.at[idx_v.at[j]],
            rows_v.at[pl.ds(j * CHUNK, CHUNK)],
            sem,
        ).wait()
        return carry

    lax.fori_loop(0, NCH, body, 0)
    pltpu.sync_copy(rows_v, out_hbm.at[pl.ds(wid * EPW, EPW)])


def _scatter_mean_parts_body(tp_hbm, idx2d_hbm, idxflat_hbm, psum_hbm,
                             pcnt_hbm, idx_v, idxf_v, tp_v, cnt_v, acc_sh):
    cid = lax.axis_index("c")
    sid = lax.axis_index("s")
    wid = sid * NC + cid
    z16 = jnp.zeros((16,), jnp.float32)
    ones16 = jnp.ones((16,), jnp.float32)

    def zero_body(i, carry):
        tp_v[i, :] = z16
        cnt_v[pl.ds(i * 16, 16)] = z16
        return carry

    lax.fori_loop(0, ROWS_PER_TILE, zero_body, 0)
    pltpu.sync_copy(tp_v.at[pl.ds(0, ROWS_PER_TILE)],
                    acc_sh.at[pl.ds(sid * ROWS_PER_TILE, ROWS_PER_TILE)])
    plsc.subcore_barrier()

    pltpu.sync_copy(idx2d_hbm.at[pl.ds(wid * NCH, NCH)], idx_v)
    pltpu.sync_copy(idxflat_hbm.at[pl.ds(wid * EPW, EPW)], idxf_v)
    pltpu.sync_copy(tp_hbm.at[pl.ds(wid * EPW, EPW)], tp_v)

    def hist_body(i, carry):
        idxs = idxf_v[pl.ds(i * 16, 16)]
        plsc.addupdate_scatter(cnt_v, [idxs], ones16)
        return carry

    lax.fori_loop(0, EPW // 16, hist_body, 0)
    n_tail = EPW - (EPW // 16) * 16
    if n_tail:
        idxs = idxf_v[pl.ds(EPW - 16, 16)]
        tail_mask = lax.iota(jnp.int32, 16) >= (16 - n_tail)
        plsc.addupdate_scatter(cnt_v, [idxs], ones16, mask=tail_mask)

    def scat_body(j, carry):
        pltpu.sync_copy(tp_v.at[pl.ds(j * CHUNK, CHUNK)],
                        acc_sh.at[idx_v.at[j]], add=True)
        return carry

    lax.fori_loop(0, NCH, scat_body, 0)
    plsc.subcore_barrier()

    pltpu.sync_copy(acc_sh.at[pl.ds(sid * ROWS_PER_TILE, ROWS_PER_TILE)],
                    psum_hbm.at[cid].at[pl.ds(sid * ROWS_PER_TILE, ROWS_PER_TILE)])
    pltpu.sync_copy(cnt_v, pcnt_hbm.at[wid])


BE = 2000


def _edge_body(ea_ref, x_ref, sh_ref, w1_ref, b1_ref, w2r_ref, bm_ref, tp_ref):
    ea = ea_ref[...]
    xs = x_ref[...] * (sh_ref[...] * ALPHA)
    h = jnp.maximum(
        jnp.dot(ea, w1_ref[...], preferred_element_type=jnp.float32,
                precision=lax.Precision.HIGHEST) + b1_ref[...], 0.0)
    z = (h[:, :, None] * xs[:, None, :]).reshape(BE, HID * C)
    tp = jnp.dot(z, w2r_ref[...], preferred_element_type=jnp.float32,
                 precision=lax.Precision.HIGHEST)
    tp += jnp.dot(xs, bm_ref[...], preferred_element_type=jnp.float32,
                  precision=lax.Precision.HIGHEST)
    tp_ref[...] = tp


_edge_tc = pl.pallas_call(
    _edge_body,
    out_shape=jax.ShapeDtypeStruct((E, D), jnp.float32),
    grid=(E // BE,),
    in_specs=[
        pl.BlockSpec((BE, NEF), lambda i: (i, 0)),
        pl.BlockSpec((BE, C), lambda i: (i, 0)),
        pl.BlockSpec((BE, 1), lambda i: (i, 0)),
        pl.BlockSpec((NEF, HID), lambda i: (0, 0)),
        pl.BlockSpec((1, HID), lambda i: (0, 0)),
        pl.BlockSpec((HID * C, D), lambda i: (0, 0)),
        pl.BlockSpec((C, D), lambda i: (0, 0)),
    ],
    out_specs=pl.BlockSpec((BE, D), lambda i: (i, 0)),
    compiler_params=pltpu.CompilerParams(
        dimension_semantics=("arbitrary",)),
)


def _fin_body(psum_ref, pcnt_ref, na_ref, bnw_ref, bnb_ref, out_ref):
    s = psum_ref[0] + psum_ref[1]
    cnt = jnp.sum(pcnt_ref[...], axis=0)
    o = s / jnp.maximum(cnt, 1.0)[:, None] + na_ref[...]
    m = jnp.mean(o, axis=0, keepdims=True)
    v = jnp.mean((o - m) ** 2, axis=0, keepdims=True)
    out_ref[...] = (o - m) * lax.rsqrt(v + 1e-5) * bnw_ref[...] + bnb_ref[...]


_finalize_tc = pl.pallas_call(
    _fin_body,
    out_shape=jax.ShapeDtypeStruct((N, D), jnp.float32),
)


@functools.lru_cache(maxsize=1)
def _build_sc_kernels():
    mesh = plsc.VectorSubcoreMesh(core_axis_name="c", subcore_axis_name="s")
    sc_params = pltpu.CompilerParams(use_tc_tiling_on_sc=False,
                                     needs_layout_passes=False)
    gather = pl.kernel(
        _gather_rows_body,
        out_type=jax.ShapeDtypeStruct((E, C), jnp.float32),
        mesh=mesh,
        compiler_params=sc_params,
        scratch_types=[
            pltpu.VMEM((NCH, CHUNK), jnp.int32),
            pltpu.VMEM((EPW, C), jnp.float32),
            pltpu.SemaphoreType.DMA,
        ],
    )
    scatter = pl.kernel(
        _scatter_mean_parts_body,
        out_type=(
            jax.ShapeDtypeStruct((NC, N, D), jnp.float32),
            jax.ShapeDtypeStruct((NW, N), jnp.float32),
        ),
        mesh=mesh,
        compiler_params=sc_params,
        scratch_types=[
            pltpu.VMEM((NCH, CHUNK), jnp.int32),
            pltpu.VMEM((EPW,), jnp.int32),
            pltpu.VMEM((EPW, D), jnp.float32),
            pltpu.VMEM((N,), jnp.float32),
            pltpu.VMEM_SHARED((N, D), jnp.float32),
        ],
    )
    return gather, scatter


def kernel(node_attr, edge_index, edge_attr, edge_sh, global_graph_embedding,
           ptr, W1, b1, W2, b2, bn_w, bn_b):
    _gather_rows, _scatter_mean_parts = _build_sc_kernels()
    edge_src = edge_index[0]
    edge_dst = edge_index[1]
    dst2d = edge_dst.reshape(NW * NCH, CHUNK)
    src2d = edge_src.reshape(NW * NCH, CHUNK)

    x = _gather_rows(node_attr, dst2d)

    w2r = W2.reshape(HID, C, D).reshape(HID * C, D)
    bm = b2.reshape(C, D)
    tp = _edge_tc(edge_attr, x, edge_sh, W1, b1.reshape(1, HID), w2r, bm)

    psum, pcnt = _scatter_mean_parts(tp, src2d, edge_src)

    return _finalize_tc(psum, pcnt, node_attr, bn_w.reshape(1, D),
                        bn_b.reshape(1, D))


# speedup vs baseline: 3.8456x; 2.9681x over previous
import functools

import jax
import jax.numpy as jnp
import numpy as np
from jax import lax
from jax.experimental import pallas as pl
from jax.experimental.pallas import tpu as pltpu
from jax.experimental.pallas import tpu_sc as plsc

N = 10000
E = 160000
C = 16
D = 16
NEF = 16
HID = 16
ALPHA = 1.0 / np.sqrt(C * 1)

NC = 2
NS = 16
NW = NC * NS
EPW = E // NW
CHUNK = 125
NCH = EPW // CHUNK
ROWS_PER_TILE = N // NS

def _gather_rows_body(table_hbm, idx_hbm, out_hbm, idx_v, rows_v, sem):
    cid = lax.axis_index("c")
    sid = lax.axis_index("s")
    wid = sid * NC + cid
    pltpu.sync_copy(idx_hbm.at[pl.ds(wid * NCH, NCH)], idx_v)

    def body(j, carry):
        pltpu.async_copy(
            table_hbm.at[idx_v.at[j]],
            rows_v.at[pl.ds(j * CHUNK, CHUNK)],
            sem,
        ).wait()
        return carry

    lax.fori_loop(0, NCH, body, 0)
    pltpu.sync_copy(rows_v, out_hbm.at[pl.ds(wid * EPW, EPW)])


def _scatter_mean_parts_body(tp_hbm, idx2d_hbm, idxflat_hbm, psum_hbm,
                             pcnt_hbm, idx_v, idxf_v, tp_v, cnt_v, acc_sh):
    cid = lax.axis_index("c")
    sid = lax.axis_index("s")
    wid = sid * NC + cid
    z16 = jnp.zeros((16,), jnp.float32)
    ones16 = jnp.ones((16,), jnp.float32)

    def zero_body(i, carry):
        tp_v[i, :] = z16
        cnt_v[pl.ds(i * 16, 16)] = z16
        return carry

    lax.fori_loop(0, ROWS_PER_TILE, zero_body, 0)
    pltpu.sync_copy(tp_v.at[pl.ds(0, ROWS_PER_TILE)],
                    acc_sh.at[pl.ds(sid * ROWS_PER_TILE, ROWS_PER_TILE)])
    plsc.subcore_barrier()

    pltpu.sync_copy(idx2d_hbm.at[pl.ds(wid * NCH, NCH)], idx_v)
    pltpu.sync_copy(idxflat_hbm.at[pl.ds(wid * EPW, EPW)], idxf_v)
    pltpu.sync_copy(tp_hbm.at[pl.ds(wid * EPW, EPW)], tp_v)

    def hist_body(i, carry):
        idxs = idxf_v[pl.ds(i * 16, 16)]
        plsc.addupdate_scatter(cnt_v, [idxs], ones16)
        return carry

    lax.fori_loop(0, EPW // 16, hist_body, 0)
    n_tail = EPW - (EPW // 16) * 16
    if n_tail:
        idxs = idxf_v[pl.ds(EPW - 16, 16)]
        tail_mask = lax.iota(jnp.int32, 16) >= (16 - n_tail)
        plsc.addupdate_scatter(cnt_v, [idxs], ones16, mask=tail_mask)

    def scat_body(j, carry):
        pltpu.sync_copy(tp_v.at[pl.ds(j * CHUNK, CHUNK)],
                        acc_sh.at[idx_v.at[j]], add=True)
        return carry

    lax.fori_loop(0, NCH, scat_body, 0)
    plsc.subcore_barrier()

    pltpu.sync_copy(acc_sh.at[pl.ds(sid * ROWS_PER_TILE, ROWS_PER_TILE)],
                    psum_hbm.at[cid].at[pl.ds(sid * ROWS_PER_TILE, ROWS_PER_TILE)])
    pltpu.sync_copy(cnt_v, pcnt_hbm.at[wid])


BE = 4000
KC = HID * C


def _edge_body(ea_ref, x_ref, sh_ref, w1_ref, b1_ref, m_ref, w2r_ref, tp_ref):
    ea = ea_ref[...]
    xs = x_ref[...] * (sh_ref[...] * ALPHA)
    h = jnp.maximum(
        jnp.dot(ea, w1_ref[...], preferred_element_type=jnp.float32)
        + b1_ref[...], 0.0)
    g = jnp.concatenate([h, xs], axis=1)
    y = jnp.dot(g, m_ref[...], preferred_element_type=jnp.float32)
    z = y[:, :KC] * y[:, KC:2 * KC]
    tp = jnp.dot(z, w2r_ref[...], preferred_element_type=jnp.float32)
    tp_ref[...] = tp + y[:, 2 * KC:]


_edge_tc = pl.pallas_call(
    _edge_body,
    out_shape=jax.ShapeDtypeStruct((E, D), jnp.float32),
    grid=(E // BE,),
    in_specs=[
        pl.BlockSpec((BE, NEF), lambda i: (i, 0)),
        pl.BlockSpec((BE, C), lambda i: (i, 0)),
        pl.BlockSpec((BE, 1), lambda i: (i, 0)),
        pl.BlockSpec((NEF, HID), lambda i: (0, 0)),
        pl.BlockSpec((1, HID), lambda i: (0, 0)),
        pl.BlockSpec((2 * HID, 2 * KC + D), lambda i: (0, 0)),
        pl.BlockSpec((KC, D), lambda i: (0, 0)),
    ],
    out_specs=pl.BlockSpec((BE, D), lambda i: (i, 0)),
    compiler_params=pltpu.CompilerParams(
        dimension_semantics=("arbitrary",)),
)

_RH = np.kron(np.eye(HID, dtype=np.float32), np.ones((1, C), np.float32))
_RX = np.tile(np.eye(C, dtype=np.float32), (1, HID))


def _fin_body(psum_ref, pcnt_ref, na_ref, bnw_ref, bnb_ref, out_ref):
    s = psum_ref[0] + psum_ref[1]
    cnt = jnp.sum(pcnt_ref[...], axis=0)
    o = s / jnp.maximum(cnt, 1.0)[:, None] + na_ref[...]
    m = jnp.mean(o, axis=0, keepdims=True)
    v = jnp.mean((o - m) ** 2, axis=0, keepdims=True)
    out_ref[...] = (o - m) * lax.rsqrt(v + 1e-5) * bnw_ref[...] + bnb_ref[...]


_finalize_tc = pl.pallas_call(
    _fin_body,
    out_shape=jax.ShapeDtypeStruct((N, D), jnp.float32),
)


@functools.lru_cache(maxsize=1)
def _build_sc_kernels():
    mesh = plsc.VectorSubcoreMesh(core_axis_name="c", subcore_axis_name="s")
    sc_params = pltpu.CompilerParams(use_tc_tiling_on_sc=False,
                                     needs_layout_passes=False)
    gather = pl.kernel(
        _gather_rows_body,
        out_type=jax.ShapeDtypeStruct((E, C), jnp.float32),
        mesh=mesh,
        compiler_params=sc_params,
        scratch_types=[
            pltpu.VMEM((NCH, CHUNK), jnp.int32),
            pltpu.VMEM((EPW, C), jnp.float32),
            pltpu.SemaphoreType.DMA,
        ],
    )
    scatter = pl.kernel(
        _scatter_mean_parts_body,
        out_type=(
            jax.ShapeDtypeStruct((NC, N, D), jnp.float32),
            jax.ShapeDtypeStruct((NW, N), jnp.float32),
        ),
        mesh=mesh,
        compiler_params=sc_params,
        scratch_types=[
            pltpu.VMEM((NCH, CHUNK), jnp.int32),
            pltpu.VMEM((EPW,), jnp.int32),
            pltpu.VMEM((EPW, D), jnp.float32),
            pltpu.VMEM((N,), jnp.float32),
            pltpu.VMEM_SHARED((N, D), jnp.float32),
        ],
    )
    return gather, scatter


def kernel(node_attr, edge_index, edge_attr, edge_sh, global_graph_embedding,
           ptr, W1, b1, W2, b2, bn_w, bn_b):
    _gather_rows, _scatter_mean_parts = _build_sc_kernels()
    edge_src = edge_index[0]
    edge_dst = edge_index[1]
    dst2d = edge_dst.reshape(NW * NCH, CHUNK)
    src2d = edge_src.reshape(NW * NCH, CHUNK)

    x = _gather_rows(node_attr, dst2d)

    w2r = W2.reshape(HID, C, D).reshape(HID * C, D)
    bm = b2.reshape(C, D)
    m = jnp.concatenate([
        jnp.concatenate([_RH, jnp.zeros((HID, KC + D), jnp.float32)], axis=1),
        jnp.concatenate([jnp.zeros((C, KC), jnp.float32), _RX, bm], axis=1),
    ], axis=0)
    tp = _edge_tc(edge_attr, x, edge_sh, W1, b1.reshape(1, HID), m, w2r)

    psum, pcnt = _scatter_mean_parts(tp, src2d, edge_src)

    return _finalize_tc(psum, pcnt, node_attr, bn_w.reshape(1, D),
                        bn_b.reshape(1, D))


# speedup vs baseline: 5.1372x; 1.3359x over previous
import functools

import jax
import jax.numpy as jnp
import numpy as np
from jax import lax
from jax.experimental import pallas as pl
from jax.experimental.pallas import tpu as pltpu
from jax.experimental.pallas import tpu_sc as plsc

N = 10000
E = 160000
C = 16
D = 16
NEF = 16
HID = 16
ALPHA = 1.0 / np.sqrt(C * 1)

NC = 2
NS = 16
NW = NC * NS
EPW = E // NW
CHUNK = 125
NCH = EPW // CHUNK
ROWS_PER_TILE = N // NS

Q = E // 8


def _gather_rows_body(table_hbm, idx_hbm, out_hbm, idx_v, rows_v, sem):
    cid = lax.axis_index("c")
    sid = lax.axis_index("s")
    wid = sid * NC + cid
    lane_j = wid // 4
    row_0 = (wid % 4) * EPW
    pltpu.sync_copy(idx_hbm.at[pl.ds(wid * NCH, NCH)], idx_v)

    def fire(j, carry):
        pltpu.async_copy(
            table_hbm.at[idx_v.at[j]],
            rows_v.at[pl.ds(j * CHUNK, CHUNK)],
            sem,
        )
        return carry

    lax.fori_loop(0, NCH, fire, 0)

    def drain(j, carry):
        pltpu.make_async_copy(
            table_hbm.at[idx_v.at[j]],
            rows_v.at[pl.ds(j * CHUNK, CHUNK)],
            sem,
        ).wait()
        return carry

    lax.fori_loop(0, NCH, drain, 0)
    pltpu.sync_copy(rows_v,
                    out_hbm.at[pl.ds(row_0, EPW), pl.ds(16 * lane_j, 16)])


def _scatter_mean_parts_body(tp_hbm, idx2d_hbm, idxflat_hbm, psum_hbm,
                             pcnt_hbm, idx_v, idxf_v, tp_v, cnt_v, acc_sh):
    cid = lax.axis_index("c")
    sid = lax.axis_index("s")
    wid = sid * NC + cid
    lane_j = wid // 4
    row_0 = (wid % 4) * EPW
    z16 = jnp.zeros((16,), jnp.float32)
    ones16 = jnp.ones((16,), jnp.float32)

    def zero_body(i, carry):
        tp_v[i, :] = z16
        cnt_v[pl.ds(i * 16, 16)] = z16
        return carry

    lax.fori_loop(0, ROWS_PER_TILE, zero_body, 0)
    pltpu.sync_copy(tp_v.at[pl.ds(0, ROWS_PER_TILE)],
                    acc_sh.at[pl.ds(sid * ROWS_PER_TILE, ROWS_PER_TILE)])
    plsc.subcore_barrier()

    pltpu.sync_copy(idx2d_hbm.at[pl.ds(wid * NCH, NCH)], idx_v)
    pltpu.sync_copy(idxflat_hbm.at[pl.ds(wid * EPW, EPW)], idxf_v)
    pltpu.sync_copy(tp_hbm.at[pl.ds(row_0, EPW), pl.ds(16 * lane_j, 16)],
                    tp_v)

    def hist_body(i, carry):
        idxs = idxf_v[pl.ds(i * 16, 16)]
        plsc.addupdate_scatter(cnt_v, [idxs], ones16)
        return carry

    lax.fori_loop(0, EPW // 16, hist_body, 0)
    n_tail = EPW - (EPW // 16) * 16
    if n_tail:
        idxs = idxf_v[pl.ds(EPW - 16, 16)]
        tail_mask = lax.iota(jnp.int32, 16) >= (16 - n_tail)
        plsc.addupdate_scatter(cnt_v, [idxs], ones16, mask=tail_mask)

    def scat_body(j, carry):
        pltpu.sync_copy(tp_v.at[pl.ds(j * CHUNK, CHUNK)],
                        acc_sh.at[idx_v.at[j]], add=True)
        return carry

    lax.fori_loop(0, NCH, scat_body, 0)
    plsc.subcore_barrier()

    pltpu.sync_copy(acc_sh.at[pl.ds(sid * ROWS_PER_TILE, ROWS_PER_TILE)],
                    psum_hbm.at[cid].at[pl.ds(sid * ROWS_PER_TILE, ROWS_PER_TILE)])
    pltpu.sync_copy(cnt_v, pcnt_hbm.at[wid])


BLK = 1000
NBLK = Q // BLK
KC = HID * C


def _edge_body(x_ref, *refs):
    ea_refs = refs[0:8]
    sh_refs = refs[8:16]
    w1_ref, b1_ref, m_ref, w2r_ref, tp_ref = refs[16:]
    w1 = w1_ref[...]
    b1 = b1_ref[...]
    m = m_ref[...]
    w2r = w2r_ref[...]
    for j in range(8):
        xs = x_ref[:, 16 * j:16 * (j + 1)] * (sh_refs[j][...] * ALPHA)
        h = jnp.maximum(
            jnp.dot(ea_refs[j][...], w1,
                    preferred_element_type=jnp.float32) + b1, 0.0)
        g = jnp.concatenate([h, xs], axis=1)
        y = jnp.dot(g, m, preferred_element_type=jnp.float32)
        z = y[:, :KC] * y[:, KC:2 * KC]
        tp = jnp.dot(z, w2r, preferred_element_type=jnp.float32)
        tp_ref[:, 16 * j:16 * (j + 1)] = tp + y[:, 2 * KC:]


_edge_tc = pl.pallas_call(
    _edge_body,
    out_shape=jax.ShapeDtypeStruct((Q, 128), jnp.float32),
    grid=(NBLK,),
    in_specs=(
        [pl.BlockSpec((BLK, 128), lambda i: (i, 0))]
        + [pl.BlockSpec((BLK, NEF), lambda i, j=j: (j * NBLK + i, 0))
           for j in range(8)]
        + [pl.BlockSpec((BLK, 1), lambda i, j=j: (j * NBLK + i, 0))
           for j in range(8)]
        + [
            pl.BlockSpec((NEF, HID), lambda i: (0, 0)),
            pl.BlockSpec((1, HID), lambda i: (0, 0)),
            pl.BlockSpec((2 * HID, 2 * KC + D), lambda i: (0, 0)),
            pl.BlockSpec((KC, D), lambda i: (0, 0)),
        ]
    ),
    out_specs=pl.BlockSpec((BLK, 128), lambda i: (i, 0)),
    compiler_params=pltpu.CompilerParams(
        dimension_semantics=("arbitrary",)),
)

_RH = np.kron(np.eye(HID, dtype=np.float32), np.ones((1, C), np.float32))
_RX = np.tile(np.eye(C, dtype=np.float32), (1, HID))


def _fin_body(psum_ref, pcnt_ref, na_ref, bnw_ref, bnb_ref, out_ref):
    s = psum_ref[0] + psum_ref[1]
    cnt = jnp.sum(pcnt_ref[...], axis=0)
    o = s / jnp.maximum(cnt, 1.0)[:, None] + na_ref[...]
    m = jnp.mean(o, axis=0, keepdims=True)
    v = jnp.mean((o - m) ** 2, axis=0, keepdims=True)
    out_ref[...] = (o - m) * lax.rsqrt(v + 1e-5) * bnw_ref[...] + bnb_ref[...]


_finalize_tc = pl.pallas_call(
    _fin_body,
    out_shape=jax.ShapeDtypeStruct((N, D), jnp.float32),
)


@functools.lru_cache(maxsize=1)
def _build_sc_kernels():
    mesh = plsc.VectorSubcoreMesh(core_axis_name="c", subcore_axis_name="s")
    sc_params = pltpu.CompilerParams(use_tc_tiling_on_sc=False,
                                     needs_layout_passes=False)
    gather = pl.kernel(
        _gather_rows_body,
        out_type=jax.ShapeDtypeStruct((Q, 128), jnp.float32),
        mesh=mesh,
        compiler_params=sc_params,
        scratch_types=[
            pltpu.VMEM((NCH, CHUNK), jnp.int32),
            pltpu.VMEM((EPW, C), jnp.float32),
            pltpu.SemaphoreType.DMA,
        ],
    )
    scatter = pl.kernel(
        _scatter_mean_parts_body,
        out_type=(
            jax.ShapeDtypeStruct((NC, N, D), jnp.float32),
            jax.ShapeDtypeStruct((NW, N), jnp.float32),
        ),
        mesh=mesh,
        compiler_params=sc_params,
        scratch_types=[
            pltpu.VMEM((NCH, CHUNK), jnp.int32),
            pltpu.VMEM((EPW,), jnp.int32),
            pltpu.VMEM((EPW, D), jnp.float32),
            pltpu.VMEM((N,), jnp.float32),
            pltpu.VMEM_SHARED((N, D), jnp.float32),
        ],
    )
    return gather, scatter


def kernel(node_attr, edge_index, edge_attr, edge_sh, global_graph_embedding,
           ptr, W1, b1, W2, b2, bn_w, bn_b):
    _gather_rows, _scatter_mean_parts = _build_sc_kernels()
    edge_src = edge_index[0]
    edge_dst = edge_index[1]
    dst2d = edge_dst.reshape(NW * NCH, CHUNK)
    src2d = edge_src.reshape(NW * NCH, CHUNK)

    x = _gather_rows(node_attr, dst2d)

    w2r = W2.reshape(HID, C, D).reshape(HID * C, D)
    bm = b2.reshape(C, D)
    m = jnp.concatenate([
        jnp.concatenate([_RH, jnp.zeros((HID, KC + D), jnp.float32)], axis=1),
        jnp.concatenate([jnp.zeros((C, KC), jnp.float32), _RX, bm], axis=1),
    ], axis=0)
    tp128 = _edge_tc(x, *([edge_attr] * 8), *([edge_sh] * 8), W1,
                     b1.reshape(1, HID), m, w2r)

    psum, pcnt = _scatter_mean_parts(tp128, src2d, edge_src)

    return _finalize_tc(psum, pcnt, node_attr, bn_w.reshape(1, D),
                        bn_b.reshape(1, D))


# speedup vs baseline: 6.2296x; 1.2126x over previous
import functools

import jax
import jax.numpy as jnp
import numpy as np
from jax import lax
from jax.experimental import pallas as pl
from jax.experimental.pallas import tpu as pltpu
from jax.experimental.pallas import tpu_sc as plsc

N = 10000
E = 160000
C = 16
D = 16
NEF = 16
HID = 16
ALPHA = 1.0 / np.sqrt(C * 1)

NC = 2
NS = 16
NW = NC * NS
EPW = E // NW
CHUNK = 125
NCH = EPW // CHUNK
ROWS_PER_TILE = N // NS

Q = E // 8


def _gather_rows_body(table_hbm, idx_hbm, out_hbm, idx_v, rows_v, sem):
    cid = lax.axis_index("c")
    sid = lax.axis_index("s")
    wid = sid * NC + cid
    lane_j = wid // 4
    row_0 = (wid % 4) * EPW
    pltpu.sync_copy(idx_hbm.at[pl.ds(wid * NCH, NCH)], idx_v)

    def fire(j, carry):
        pltpu.async_copy(
            table_hbm.at[idx_v.at[j]],
            rows_v.at[pl.ds(j * CHUNK, CHUNK)],
            sem,
        )
        return carry

    lax.fori_loop(0, NCH, fire, 0)

    def drain(j, carry):
        pltpu.make_async_copy(
            table_hbm.at[idx_v.at[j]],
            rows_v.at[pl.ds(j * CHUNK, CHUNK)],
            sem,
        ).wait()
        return carry

    lax.fori_loop(0, NCH, drain, 0)
    pltpu.sync_copy(rows_v,
                    out_hbm.at[pl.ds(row_0, EPW), pl.ds(16 * lane_j, 16)])


def _scatter_mean_parts_body(tp_hbm, idx2d_hbm, idxflat_hbm, psum_hbm,
                             pcnt_hbm, idx_v, idxf_v, tp_v, cnt_v, acc_sh):
    cid = lax.axis_index("c")
    sid = lax.axis_index("s")
    wid = sid * NC + cid
    lane_j = wid // 4
    row_0 = (wid % 4) * EPW
    z16 = jnp.zeros((16,), jnp.float32)
    ones16 = jnp.ones((16,), jnp.float32)

    def zero_body(i, carry):
        tp_v[i, :] = z16
        cnt_v[pl.ds(i * 16, 16)] = z16
        return carry

    lax.fori_loop(0, ROWS_PER_TILE, zero_body, 0)
    pltpu.sync_copy(tp_v.at[pl.ds(0, ROWS_PER_TILE)],
                    acc_sh.at[pl.ds(sid * ROWS_PER_TILE, ROWS_PER_TILE)])
    plsc.subcore_barrier()

    pltpu.sync_copy(idx2d_hbm.at[pl.ds(wid * NCH, NCH)], idx_v)
    pltpu.sync_copy(idxflat_hbm.at[pl.ds(wid * EPW, EPW)], idxf_v)
    pltpu.sync_copy(tp_hbm.at[pl.ds(row_0, EPW), pl.ds(16 * lane_j, 16)],
                    tp_v)

    def hist_body(i, carry):
        idxs = idxf_v[pl.ds(i * 16, 16)]
        plsc.addupdate_scatter(cnt_v, [idxs], ones16)
        return carry

    lax.fori_loop(0, EPW // 16, hist_body, 0)
    n_tail = EPW - (EPW // 16) * 16
    if n_tail:
        idxs = idxf_v[pl.ds(EPW - 16, 16)]
        tail_mask = lax.iota(jnp.int32, 16) >= (16 - n_tail)
        plsc.addupdate_scatter(cnt_v, [idxs], ones16, mask=tail_mask)

    def scat_body(j, carry):
        pltpu.sync_copy(tp_v.at[pl.ds(j * CHUNK, CHUNK)],
                        acc_sh.at[idx_v.at[j]], add=True)
        return carry

    lax.fori_loop(0, NCH, scat_body, 0)
    plsc.subcore_barrier()

    pltpu.sync_copy(acc_sh.at[pl.ds(sid * ROWS_PER_TILE, ROWS_PER_TILE)],
                    psum_hbm.at[cid].at[pl.ds(sid * ROWS_PER_TILE, ROWS_PER_TILE)])
    pltpu.sync_copy(cnt_v, pcnt_hbm.at[wid])


BLK = 1000
NBLK = Q // BLK
KC = HID * C


def _edge_body(x_ref, *refs):
    ea_refs = refs[0:8]
    sht_ref = refs[8]
    w1_ref, b1_ref, m3_ref, w2r_ref, rs8_ref, tp_ref = refs[9:]
    w1 = w1_ref[...]
    b1 = b1_ref[...]
    w2r = w2r_ref[...]
    xf = x_ref[...]
    x_stack = jnp.concatenate(
        [xf[:, 16 * j:16 * (j + 1)] for j in range(8)], axis=0)
    ea_all = jnp.concatenate([r[...] for r in ea_refs], axis=0)
    h = jnp.maximum(
        jnp.dot(ea_all, w1, preferred_element_type=jnp.float32) + b1, 0.0)
    g = jnp.concatenate([h, x_stack], axis=1)
    y = jnp.dot(g, m3_ref[...], preferred_element_type=jnp.float32)
    z = y[:, :KC] * y[:, KC:2 * KC]
    tp = jnp.dot(z, w2r, preferred_element_type=jnp.float32) + y[:, 2 * KC:]
    tp_all = jnp.concatenate(
        [tp[j * BLK:(j + 1) * BLK, :] for j in range(8)], axis=1)
    sh8t = sht_ref[0]
    sh_all = jnp.dot(sh8t.T, rs8_ref[...],
                     preferred_element_type=jnp.float32)
    tp_ref[...] = tp_all * (sh_all * ALPHA)


_edge_tc = pl.pallas_call(
    _edge_body,
    out_shape=jax.ShapeDtypeStruct((Q, 128), jnp.float32),
    grid=(NBLK,),
    in_specs=(
        [pl.BlockSpec((BLK, 128), lambda i: (i, 0))]
        + [pl.BlockSpec((BLK, NEF), lambda i, j=j: (j * NBLK + i, 0))
           for j in range(8)]
        + [pl.BlockSpec((1, 8, BLK), lambda i: (i, 0, 0))]
        + [
            pl.BlockSpec((NEF, HID), lambda i: (0, 0)),
            pl.BlockSpec((1, HID), lambda i: (0, 0)),
            pl.BlockSpec((2 * HID, 2 * KC + D), lambda i: (0, 0)),
            pl.BlockSpec((KC, D), lambda i: (0, 0)),
            pl.BlockSpec((8, 128), lambda i: (0, 0)),
        ]
    ),
    out_specs=pl.BlockSpec((BLK, 128), lambda i: (i, 0)),
    compiler_params=pltpu.CompilerParams(
        dimension_semantics=("arbitrary",)),
)

_RS8 = np.kron(np.eye(8, dtype=np.float32), np.ones((1, 16), np.float32))

_RH = np.kron(np.eye(HID, dtype=np.float32), np.ones((1, C), np.float32))
_RX = np.tile(np.eye(C, dtype=np.float32), (1, HID))


def _fin_body(psum_ref, pcnt_ref, na_ref, bnw_ref, bnb_ref, out_ref):
    s = psum_ref[0] + psum_ref[1]
    cnt = jnp.sum(pcnt_ref[...], axis=0)
    o = s / jnp.maximum(cnt, 1.0)[:, None] + na_ref[...]
    m = jnp.mean(o, axis=0, keepdims=True)
    v = jnp.mean((o - m) ** 2, axis=0, keepdims=True)
    out_ref[...] = (o - m) * lax.rsqrt(v + 1e-5) * bnw_ref[...] + bnb_ref[...]


_finalize_tc = pl.pallas_call(
    _fin_body,
    out_shape=jax.ShapeDtypeStruct((N, D), jnp.float32),
)


@functools.lru_cache(maxsize=1)
def _build_sc_kernels():
    mesh = plsc.VectorSubcoreMesh(core_axis_name="c", subcore_axis_name="s")
    sc_params = pltpu.CompilerParams(use_tc_tiling_on_sc=False,
                                     needs_layout_passes=False)
    gather = pl.kernel(
        _gather_rows_body,
        out_type=jax.ShapeDtypeStruct((Q, 128), jnp.float32),
        mesh=mesh,
        compiler_params=sc_params,
        scratch_types=[
            pltpu.VMEM((NCH, CHUNK), jnp.int32),
            pltpu.VMEM((EPW, C), jnp.float32),
            pltpu.SemaphoreType.DMA,
        ],
    )
    scatter = pl.kernel(
        _scatter_mean_parts_body,
        out_type=(
            jax.ShapeDtypeStruct((NC, N, D), jnp.float32),
            jax.ShapeDtypeStruct((NW, N), jnp.float32),
        ),
        mesh=mesh,
        compiler_params=sc_params,
        scratch_types=[
            pltpu.VMEM((NCH, CHUNK), jnp.int32),
            pltpu.VMEM((EPW,), jnp.int32),
            pltpu.VMEM((EPW, D), jnp.float32),
            pltpu.VMEM((N,), jnp.float32),
            pltpu.VMEM_SHARED((N, D), jnp.float32),
        ],
    )
    return gather, scatter


def kernel(node_attr, edge_index, edge_attr, edge_sh, global_graph_embedding,
           ptr, W1, b1, W2, b2, bn_w, bn_b):
    _gather_rows, _scatter_mean_parts = _build_sc_kernels()
    edge_src = edge_index[0]
    edge_dst = edge_index[1]
    dst2d = edge_dst.reshape(NW * NCH, CHUNK)
    src2d = edge_src.reshape(NW * NCH, CHUNK)

    x = _gather_rows(node_attr, dst2d)

    w2r = W2.reshape(HID, C, D).reshape(HID * C, D)
    bm = b2.reshape(C, D)
    m3 = jnp.concatenate([
        jnp.concatenate([_RH, jnp.zeros((HID, KC + D), jnp.float32)], axis=1),
        jnp.concatenate([jnp.zeros((C, KC), jnp.float32), _RX, bm], axis=1),
    ], axis=0)
    sht = jnp.transpose(edge_sh.reshape(8, NBLK, BLK), (1, 0, 2))
    tp128 = _edge_tc(x, *([edge_attr] * 8), sht, W1,
                     b1.reshape(1, HID), m3, w2r, jnp.asarray(_RS8))

    psum, pcnt = _scatter_mean_parts(tp128, src2d, edge_src)

    return _finalize_tc(psum, pcnt, node_attr, bn_w.reshape(1, D),
                        bn_b.reshape(1, D))


# speedup vs baseline: 6.3067x; 1.0124x over previous
import functools

import jax
import jax.numpy as jnp
import numpy as np
from jax import lax
from jax.experimental import pallas as pl
from jax.experimental.pallas import tpu as pltpu
from jax.experimental.pallas import tpu_sc as plsc

N = 10000
E = 160000
C = 16
D = 16
NEF = 16
HID = 16
ALPHA = 1.0 / np.sqrt(C * 1)

NC = 2
NS = 16
NW = NC * NS
EPW = E // NW
CHUNK = 125
NCH = EPW // CHUNK
ROWS_PER_TILE = N // NS

Q = E // 8


def _gather_rows_body(table_hbm, idx_hbm, out_hbm, idx_v, rows_v, sem):
    cid = lax.axis_index("c")
    sid = lax.axis_index("s")
    wid = sid * NC + cid
    lane_j = wid // 4
    row_0 = (wid % 4) * EPW
    pltpu.sync_copy(idx_hbm.at[pl.ds(wid * NCH, NCH)], idx_v)

    def fire(j, carry):
        pltpu.async_copy(
            table_hbm.at[idx_v.at[j]],
            rows_v.at[pl.ds(j * CHUNK, CHUNK)],
            sem,
        )
        return carry

    lax.fori_loop(0, NCH, fire, 0)

    def drain(j, carry):
        pltpu.make_async_copy(
            table_hbm.at[idx_v.at[j]],
            rows_v.at[pl.ds(j * CHUNK, CHUNK)],
            sem,
        ).wait()
        return carry

    lax.fori_loop(0, NCH, drain, 0)
    pltpu.sync_copy(rows_v,
                    out_hbm.at[pl.ds(row_0, EPW), pl.ds(16 * lane_j, 16)])


def _scatter_mean_parts_body(tp_hbm, idx2d_hbm, idxflat_hbm, psum_hbm,
                             pcnt_hbm, idx_v, idxf_v, tp_v, cnt_v, acc_sh):
    cid = lax.axis_index("c")
    sid = lax.axis_index("s")
    wid = sid * NC + cid
    lane_j = wid // 4
    row_0 = (wid % 4) * EPW
    z16 = jnp.zeros((16,), jnp.float32)
    ones16 = jnp.ones((16,), jnp.float32)

    def zero_body(i, carry):
        tp_v[i, :] = z16
        cnt_v[pl.ds(i * 16, 16)] = z16
        return carry

    lax.fori_loop(0, ROWS_PER_TILE, zero_body, 0)
    pltpu.sync_copy(tp_v.at[pl.ds(0, ROWS_PER_TILE)],
                    acc_sh.at[pl.ds(sid * ROWS_PER_TILE, ROWS_PER_TILE)])
    plsc.subcore_barrier()

    pltpu.sync_copy(idx2d_hbm.at[pl.ds(wid * NCH, NCH)], idx_v)
    pltpu.sync_copy(idxflat_hbm.at[pl.ds(wid * EPW, EPW)], idxf_v)
    pltpu.sync_copy(tp_hbm.at[pl.ds(row_0, EPW), pl.ds(16 * lane_j, 16)],
                    tp_v)

    def hist_body(i, carry):
        idxs = idxf_v[pl.ds(i * 16, 16)]
        plsc.addupdate_scatter(cnt_v, [idxs], ones16)
        return carry

    lax.fori_loop(0, EPW // 16, hist_body, 0)
    n_tail = EPW - (EPW // 16) * 16
    if n_tail:
        idxs = idxf_v[pl.ds(EPW - 16, 16)]
        tail_mask = lax.iota(jnp.int32, 16) >= (16 - n_tail)
        plsc.addupdate_scatter(cnt_v, [idxs], ones16, mask=tail_mask)

    def scat_body(j, carry):
        pltpu.sync_copy(tp_v.at[pl.ds(j * CHUNK, CHUNK)],
                        acc_sh.at[idx_v.at[j]], add=True)
        return carry

    lax.fori_loop(0, NCH, scat_body, 0)
    plsc.subcore_barrier()

    pltpu.sync_copy(acc_sh.at[pl.ds(sid * ROWS_PER_TILE, ROWS_PER_TILE)],
                    psum_hbm.at[cid].at[pl.ds(sid * ROWS_PER_TILE, ROWS_PER_TILE)])
    pltpu.sync_copy(cnt_v, pcnt_hbm.at[wid])


BLK = 1000
NBLK = Q // BLK
KC = HID * C


def _edge_body(x_ref, *refs):
    ea_refs = refs[0:8]
    sht_ref = refs[8]
    w1_ref, b1_ref, m3_ref, w2r_ref, rs8_ref, eye_ref, tp_ref = refs[9:]
    w1 = w1_ref[...]
    b1 = b1_ref[...]
    w2r = w2r_ref[...]
    xf = x_ref[...]
    m3 = m3_ref[...]
    eye = eye_ref[...]
    dn0 = (((0,), (1,)), ((), ()))
    dnt = (((0,), (0,)), ((), ()))
    ea_all = jnp.concatenate([r[...] for r in ea_refs], axis=0)
    ht_all = jnp.maximum(
        lax.dot_general(w1, ea_all, dn0,
                        preferred_element_type=jnp.float32) + b1, 0.0)
    xt_all = jnp.concatenate(
        [lax.dot_general(eye, xf[:, 16 * j:16 * (j + 1)], dn0,
                         preferred_element_type=jnp.float32)
         for j in range(8)], axis=1)
    gt = jnp.concatenate([ht_all, xt_all], axis=0)
    yt = lax.dot_general(m3, gt, dnt,
                         preferred_element_type=jnp.float32)
    zt = yt[:KC] * yt[KC:2 * KC]
    tpt = lax.dot_general(w2r, zt, dnt,
                          preferred_element_type=jnp.float32)
    tpt = tpt + yt[2 * KC:]
    tp_all = jnp.transpose(jnp.concatenate(
        [tpt[:, j * BLK:(j + 1) * BLK] for j in range(8)], axis=0))
    sh8t = sht_ref[0]
    sh_all = jnp.dot(sh8t.T, rs8_ref[...],
                     preferred_element_type=jnp.float32)
    tp_ref[...] = tp_all * (sh_all * ALPHA)


_edge_tc = pl.pallas_call(
    _edge_body,
    out_shape=jax.ShapeDtypeStruct((Q, 128), jnp.float32),
    grid=(NBLK,),
    in_specs=(
        [pl.BlockSpec((BLK, 128), lambda i: (i, 0))]
        + [pl.BlockSpec((BLK, NEF), lambda i, j=j: (j * NBLK + i, 0))
           for j in range(8)]
        + [pl.BlockSpec((1, 8, BLK), lambda i: (i, 0, 0))]
        + [
            pl.BlockSpec((NEF, HID), lambda i: (0, 0)),
            pl.BlockSpec((HID, 1), lambda i: (0, 0)),
            pl.BlockSpec((2 * HID, 2 * KC + D), lambda i: (0, 0)),
            pl.BlockSpec((KC, D), lambda i: (0, 0)),
            pl.BlockSpec((8, 128), lambda i: (0, 0)),
            pl.BlockSpec((C, C), lambda i: (0, 0)),
        ]
    ),
    out_specs=pl.BlockSpec((BLK, 128), lambda i: (i, 0)),
    compiler_params=pltpu.CompilerParams(
        dimension_semantics=("arbitrary",)),
)

_RS8 = np.kron(np.eye(8, dtype=np.float32), np.ones((1, 16), np.float32))

_RH = np.kron(np.eye(HID, dtype=np.float32), np.ones((1, C), np.float32))
_RX = np.tile(np.eye(C, dtype=np.float32), (1, HID))


def _fin_body(psum_ref, pcnt_ref, na_ref, bnw_ref, bnb_ref, out_ref):
    s = psum_ref[0] + psum_ref[1]
    cnt = jnp.sum(pcnt_ref[...], axis=0)
    o = s / jnp.maximum(cnt, 1.0)[:, None] + na_ref[...]
    m = jnp.mean(o, axis=0, keepdims=True)
    v = jnp.mean((o - m) ** 2, axis=0, keepdims=True)
    out_ref[...] = (o - m) * lax.rsqrt(v + 1e-5) * bnw_ref[...] + bnb_ref[...]


_finalize_tc = pl.pallas_call(
    _fin_body,
    out_shape=jax.ShapeDtypeStruct((N, D), jnp.float32),
)


@functools.lru_cache(maxsize=1)
def _build_sc_kernels():
    mesh = plsc.VectorSubcoreMesh(core_axis_name="c", subcore_axis_name="s")
    sc_params = pltpu.CompilerParams(use_tc_tiling_on_sc=False,
                                     needs_layout_passes=False)
    gather = pl.kernel(
        _gather_rows_body,
        out_type=jax.ShapeDtypeStruct((Q, 128), jnp.float32),
        mesh=mesh,
        compiler_params=sc_params,
        scratch_types=[
            pltpu.VMEM((NCH, CHUNK), jnp.int32),
            pltpu.VMEM((EPW, C), jnp.float32),
            pltpu.SemaphoreType.DMA,
        ],
    )
    scatter = pl.kernel(
        _scatter_mean_parts_body,
        out_type=(
            jax.ShapeDtypeStruct((NC, N, D), jnp.float32),
            jax.ShapeDtypeStruct((NW, N), jnp.float32),
        ),
        mesh=mesh,
        compiler_params=sc_params,
        scratch_types=[
            pltpu.VMEM((NCH, CHUNK), jnp.int32),
            pltpu.VMEM((EPW,), jnp.int32),
            pltpu.VMEM((EPW, D), jnp.float32),
            pltpu.VMEM((N,), jnp.float32),
            pltpu.VMEM_SHARED((N, D), jnp.float32),
        ],
    )
    return gather, scatter


def kernel(node_attr, edge_index, edge_attr, edge_sh, global_graph_embedding,
           ptr, W1, b1, W2, b2, bn_w, bn_b):
    _gather_rows, _scatter_mean_parts = _build_sc_kernels()
    edge_src = edge_index[0]
    edge_dst = edge_index[1]
    dst2d = edge_dst.reshape(NW * NCH, CHUNK)
    src2d = edge_src.reshape(NW * NCH, CHUNK)

    x = _gather_rows(node_attr, dst2d)

    w2r = W2.reshape(HID, C, D).reshape(HID * C, D)
    bm = b2.reshape(C, D)
    m3 = jnp.concatenate([
        jnp.concatenate([_RH, jnp.zeros((HID, KC + D), jnp.float32)], axis=1),
        jnp.concatenate([jnp.zeros((C, KC), jnp.float32), _RX, bm], axis=1),
    ], axis=0)
    sht = jnp.transpose(edge_sh.reshape(8, NBLK, BLK), (1, 0, 2))
    tp128 = _edge_tc(x, *([edge_attr] * 8), sht, W1,
                     b1.reshape(HID, 1), m3, w2r,
                     jnp.asarray(_RS8), jnp.eye(C, dtype=jnp.float32))

    psum, pcnt = _scatter_mean_parts(tp128, src2d, edge_src)

    return _finalize_tc(psum, pcnt, node_attr, bn_w.reshape(1, D),
                        bn_b.reshape(1, D))


# speedup vs baseline: 6.5367x; 1.0365x over previous
import functools

import jax
import jax.numpy as jnp
import numpy as np
from jax import lax
from jax.experimental import pallas as pl
from jax.experimental.pallas import tpu as pltpu
from jax.experimental.pallas import tpu_sc as plsc

N = 10000
E = 160000
C = 16
D = 16
NEF = 16
HID = 16
ALPHA = 1.0 / np.sqrt(C * 1)

NC = 2
NS = 16
NW = NC * NS
EPW = E // NW
CHUNK = 125
NCH = EPW // CHUNK
ROWS_PER_TILE = N // NS

Q = E // 8


def _gather_rows_body(table_hbm, idx_hbm, out_hbm, idx_v, rows_v, sem):
    cid = lax.axis_index("c")
    sid = lax.axis_index("s")
    wid = sid * NC + cid
    lane_j = wid // 4
    row_0 = (wid % 4) * EPW
    pltpu.sync_copy(idx_hbm.at[pl.ds(wid * NCH, NCH)], idx_v)

    def fire(j, carry):
        pltpu.async_copy(
            table_hbm.at[idx_v.at[j]],
            rows_v.at[pl.ds(j * CHUNK, CHUNK)],
            sem,
        )
        return carry

    lax.fori_loop(0, NCH, fire, 0)

    def drain(j, carry):
        pltpu.make_async_copy(
            table_hbm.at[idx_v.at[j]],
            rows_v.at[pl.ds(j * CHUNK, CHUNK)],
            sem,
        ).wait()
        return carry

    lax.fori_loop(0, NCH, drain, 0)
    pltpu.sync_copy(rows_v,
                    out_hbm.at[pl.ds(row_0, EPW), pl.ds(16 * lane_j, 16)])


def _scatter_mean_parts_body(tp_hbm, idx2d_hbm, idxflat_hbm, psum_hbm,
                             pcnt_hbm, idx_v, idxf_v, tp_v, cnt_v, zb_v,
                             acc_sh, sem):
    cid = lax.axis_index("c")
    sid = lax.axis_index("s")
    wid = sid * NC + cid
    lane_j = wid // 4
    row_0 = (wid % 4) * EPW
    z16 = jnp.zeros((16,), jnp.float32)
    ones16 = jnp.ones((16,), jnp.float32)

    pltpu.async_copy(idx2d_hbm.at[pl.ds(wid * NCH, NCH)], idx_v, sem)
    pltpu.async_copy(idxflat_hbm.at[pl.ds(wid * EPW, EPW)], idxf_v, sem)
    pltpu.async_copy(tp_hbm.at[pl.ds(row_0, EPW), pl.ds(16 * lane_j, 16)],
                     tp_v, sem)

    def zero_body(i, carry):
        zb_v[i, :] = z16
        cnt_v[pl.ds(i * 16, 16)] = z16
        return carry

    lax.fori_loop(0, ROWS_PER_TILE, zero_body, 0)
    pltpu.sync_copy(zb_v, acc_sh.at[pl.ds(sid * ROWS_PER_TILE, ROWS_PER_TILE)])
    plsc.subcore_barrier()

    pltpu.make_async_copy(idx2d_hbm.at[pl.ds(wid * NCH, NCH)], idx_v,
                          sem).wait()
    pltpu.make_async_copy(idxflat_hbm.at[pl.ds(wid * EPW, EPW)], idxf_v,
                          sem).wait()
    pltpu.make_async_copy(
        tp_hbm.at[pl.ds(row_0, EPW), pl.ds(16 * lane_j, 16)], tp_v,
        sem).wait()

    def scat_fire(j, carry):
        pltpu.async_copy(tp_v.at[pl.ds(j * CHUNK, CHUNK)],
                         acc_sh.at[idx_v.at[j]], sem, add=True)
        return carry

    lax.fori_loop(0, NCH, scat_fire, 0)

    def hist_body(i, carry):
        idxs = idxf_v[pl.ds(i * 16, 16)]
        plsc.addupdate_scatter(cnt_v, [idxs], ones16)
        return carry

    lax.fori_loop(0, EPW // 16, hist_body, 0)
    n_tail = EPW - (EPW // 16) * 16
    if n_tail:
        idxs = idxf_v[pl.ds(EPW - 16, 16)]
        tail_mask = lax.iota(jnp.int32, 16) >= (16 - n_tail)
        plsc.addupdate_scatter(cnt_v, [idxs], ones16, mask=tail_mask)

    def scat_drain(j, carry):
        pltpu.make_async_copy(tp_v.at[pl.ds(j * CHUNK, CHUNK)],
                              acc_sh.at[idx_v.at[j]], sem).wait()
        return carry

    lax.fori_loop(0, NCH, scat_drain, 0)
    plsc.subcore_barrier()

    pltpu.sync_copy(acc_sh.at[pl.ds(sid * ROWS_PER_TILE, ROWS_PER_TILE)],
                    psum_hbm.at[cid].at[pl.ds(sid * ROWS_PER_TILE, ROWS_PER_TILE)])
    pltpu.sync_copy(cnt_v, pcnt_hbm.at[wid])


BLK = 1000
NBLK = Q // BLK
KC = HID * C


def _edge_body(x_ref, *refs):
    ea_refs = refs[0:8]
    sht_ref = refs[8]
    w1_ref, b1_ref, m3_ref, w2r_ref, rs8_ref, eye_ref, tp_ref = refs[9:]
    w1 = w1_ref[...]
    b1 = b1_ref[...]
    w2r = w2r_ref[...]
    xf = x_ref[...]
    m3 = m3_ref[...]
    eye = eye_ref[...]
    dn0 = (((0,), (1,)), ((), ()))
    dnt = (((0,), (0,)), ((), ()))
    ea_all = jnp.concatenate([r[...] for r in ea_refs], axis=0)
    ht_all = jnp.maximum(
        lax.dot_general(w1, ea_all, dn0,
                        preferred_element_type=jnp.float32) + b1, 0.0)
    xt_all = jnp.concatenate(
        [lax.dot_general(eye, xf[:, 16 * j:16 * (j + 1)], dn0,
                         preferred_element_type=jnp.float32)
         for j in range(8)], axis=1)
    gt = jnp.concatenate([ht_all, xt_all], axis=0)
    yt = lax.dot_general(m3, gt, dnt,
                         preferred_element_type=jnp.float32)
    zt = yt[:KC] * yt[KC:2 * KC]
    tpt = lax.dot_general(w2r, zt, dnt,
                          preferred_element_type=jnp.float32)
    tpt = tpt + yt[2 * KC:]
    tp_all = jnp.transpose(jnp.concatenate(
        [tpt[:, j * BLK:(j + 1) * BLK] for j in range(8)], axis=0))
    sh8t = sht_ref[0]
    sh_all = jnp.dot(sh8t.T, rs8_ref[...],
                     preferred_element_type=jnp.float32)
    tp_ref[...] = tp_all * (sh_all * ALPHA)


_edge_tc = pl.pallas_call(
    _edge_body,
    out_shape=jax.ShapeDtypeStruct((Q, 128), jnp.float32),
    grid=(NBLK,),
    in_specs=(
        [pl.BlockSpec((BLK, 128), lambda i: (i, 0))]
        + [pl.BlockSpec((BLK, NEF), lambda i, j=j: (j * NBLK + i, 0))
           for j in range(8)]
        + [pl.BlockSpec((1, 8, BLK), lambda i: (i, 0, 0))]
        + [
            pl.BlockSpec((NEF, HID), lambda i: (0, 0)),
            pl.BlockSpec((HID, 1), lambda i: (0, 0)),
            pl.BlockSpec((2 * HID, 2 * KC + D), lambda i: (0, 0)),
            pl.BlockSpec((KC, D), lambda i: (0, 0)),
            pl.BlockSpec((8, 128), lambda i: (0, 0)),
            pl.BlockSpec((C, C), lambda i: (0, 0)),
        ]
    ),
    out_specs=pl.BlockSpec((BLK, 128), lambda i: (i, 0)),
    compiler_params=pltpu.CompilerParams(
        dimension_semantics=("arbitrary",)),
)

_RS8 = np.kron(np.eye(8, dtype=np.float32), np.ones((1, 16), np.float32))

_RH = np.kron(np.eye(HID, dtype=np.float32), np.ones((1, C), np.float32))
_RX = np.tile(np.eye(C, dtype=np.float32), (1, HID))


def _fin_body(psum_ref, pcnt_ref, na_ref, bnw_ref, bnb_ref, out_ref):
    s = psum_ref[0] + psum_ref[1]
    cnt = jnp.sum(pcnt_ref[...], axis=0)
    o = s / jnp.maximum(cnt, 1.0)[:, None] + na_ref[...]
    m = jnp.mean(o, axis=0, keepdims=True)
    v = jnp.mean((o - m) ** 2, axis=0, keepdims=True)
    out_ref[...] = (o - m) * lax.rsqrt(v + 1e-5) * bnw_ref[...] + bnb_ref[...]


_finalize_tc = pl.pallas_call(
    _fin_body,
    out_shape=jax.ShapeDtypeStruct((N, D), jnp.float32),
)


@functools.lru_cache(maxsize=1)
def _build_sc_kernels():
    mesh = plsc.VectorSubcoreMesh(core_axis_name="c", subcore_axis_name="s")
    sc_params = pltpu.CompilerParams(use_tc_tiling_on_sc=False,
                                     needs_layout_passes=False)
    gather = pl.kernel(
        _gather_rows_body,
        out_type=jax.ShapeDtypeStruct((Q, 128), jnp.float32),
        mesh=mesh,
        compiler_params=sc_params,
        scratch_types=[
            pltpu.VMEM((NCH, CHUNK), jnp.int32),
            pltpu.VMEM((EPW, C), jnp.float32),
            pltpu.SemaphoreType.DMA,
        ],
    )
    scatter = pl.kernel(
        _scatter_mean_parts_body,
        out_type=(
            jax.ShapeDtypeStruct((NC, N, D), jnp.float32),
            jax.ShapeDtypeStruct((NW, N), jnp.float32),
        ),
        mesh=mesh,
        compiler_params=sc_params,
        scratch_types=[
            pltpu.VMEM((NCH, CHUNK), jnp.int32),
            pltpu.VMEM((EPW,), jnp.int32),
            pltpu.VMEM((EPW, D), jnp.float32),
            pltpu.VMEM((N,), jnp.float32),
            pltpu.VMEM((ROWS_PER_TILE, D), jnp.float32),
            pltpu.VMEM_SHARED((N, D), jnp.float32),
            pltpu.SemaphoreType.DMA,
        ],
    )
    return gather, scatter


def kernel(node_attr, edge_index, edge_attr, edge_sh, global_graph_embedding,
           ptr, W1, b1, W2, b2, bn_w, bn_b):
    _gather_rows, _scatter_mean_parts = _build_sc_kernels()
    edge_src = edge_index[0]
    edge_dst = edge_index[1]
    dst2d = edge_dst.reshape(NW * NCH, CHUNK)
    src2d = edge_src.reshape(NW * NCH, CHUNK)

    x = _gather_rows(node_attr, dst2d)

    w2r = W2.reshape(HID, C, D).reshape(HID * C, D)
    bm = b2.reshape(C, D)
    m3 = jnp.concatenate([
        jnp.concatenate([_RH, jnp.zeros((HID, KC + D), jnp.float32)], axis=1),
        jnp.concatenate([jnp.zeros((C, KC), jnp.float32), _RX, bm], axis=1),
    ], axis=0)
    sht = jnp.transpose(edge_sh.reshape(8, NBLK, BLK), (1, 0, 2))
    tp128 = _edge_tc(x, *([edge_attr] * 8), sht, W1,
                     b1.reshape(HID, 1), m3, w2r,
                     jnp.asarray(_RS8), jnp.eye(C, dtype=jnp.float32))

    psum, pcnt = _scatter_mean_parts(tp128, src2d, edge_src)

    return _finalize_tc(psum, pcnt, node_attr, bn_w.reshape(1, D),
                        bn_b.reshape(1, D))


# speedup vs baseline: 6.5385x; 1.0003x over previous
import functools

import jax
import jax.numpy as jnp
import numpy as np
from jax import lax
from jax.experimental import pallas as pl
from jax.experimental.pallas import tpu as pltpu
from jax.experimental.pallas import tpu_sc as plsc

N = 10000
E = 160000
C = 16
D = 16
NEF = 16
HID = 16
ALPHA = 1.0 / np.sqrt(C * 1)

NC = 2
NS = 16
NW = NC * NS
EPW = E // NW
CHUNK = 125
NCH = EPW // CHUNK
ROWS_PER_TILE = N // NS

Q = E // 8


def _gather_rows_body(table_hbm, idx_hbm, out_hbm, idx_v, rows_v, sem):
    cid = lax.axis_index("c")
    sid = lax.axis_index("s")
    wid = sid * NC + cid
    lane_j = wid // 4
    row_0 = (wid % 4) * EPW
    pltpu.sync_copy(idx_hbm.at[pl.ds(wid * NCH, NCH)], idx_v)

    def fire(j, carry):
        pltpu.async_copy(
            table_hbm.at[idx_v.at[j]],
            rows_v.at[pl.ds(j * CHUNK, CHUNK)],
            sem,
        )
        return carry

    lax.fori_loop(0, NCH, fire, 0)

    def drain(j, carry):
        pltpu.make_async_copy(
            table_hbm.at[idx_v.at[j]],
            rows_v.at[pl.ds(j * CHUNK, CHUNK)],
            sem,
        ).wait()
        return carry

    lax.fori_loop(0, NCH, drain, 0)
    pltpu.sync_copy(rows_v,
                    out_hbm.at[pl.ds(row_0, EPW), pl.ds(16 * lane_j, 16)])


def _scatter_mean_parts_body(tp_hbm, idx2d_hbm, idxflat_hbm, psum_hbm,
                             pcnt_hbm, idx_v, idxf_v, tp_v, cnt_v, zb_v,
                             acc_sh, sem):
    cid = lax.axis_index("c")
    sid = lax.axis_index("s")
    wid = sid * NC + cid
    lane_j = wid // 4
    row_0 = (wid % 4) * EPW
    z16 = jnp.zeros((16,), jnp.float32)
    ones16 = jnp.ones((16,), jnp.float32)

    pltpu.async_copy(idx2d_hbm.at[pl.ds(wid * NCH, NCH)], idx_v, sem)
    pltpu.async_copy(idxflat_hbm.at[pl.ds(wid * EPW, EPW)], idxf_v, sem)
    pltpu.async_copy(tp_hbm.at[pl.ds(row_0, EPW), pl.ds(16 * lane_j, 16)],
                     tp_v, sem)

    def zero_body(i, carry):
        zb_v[i, :] = z16
        cnt_v[pl.ds(i * 16, 16)] = z16
        return carry

    lax.fori_loop(0, ROWS_PER_TILE, zero_body, 0)
    pltpu.sync_copy(zb_v, acc_sh.at[pl.ds(sid * ROWS_PER_TILE, ROWS_PER_TILE)])
    plsc.subcore_barrier()

    pltpu.make_async_copy(idx2d_hbm.at[pl.ds(wid * NCH, NCH)], idx_v,
                          sem).wait()
    pltpu.make_async_copy(idxflat_hbm.at[pl.ds(wid * EPW, EPW)], idxf_v,
                          sem).wait()
    pltpu.make_async_copy(
        tp_hbm.at[pl.ds(row_0, EPW), pl.ds(16 * lane_j, 16)], tp_v,
        sem).wait()

    def scat_fire(j, carry):
        pltpu.async_copy(tp_v.at[pl.ds(j * CHUNK, CHUNK)],
                         acc_sh.at[idx_v.at[j]], sem, add=True)
        return carry

    lax.fori_loop(0, NCH, scat_fire, 0)

    def hist_body(i, carry):
        idxs = idxf_v[pl.ds(i * 16, 16)]
        plsc.addupdate_scatter(cnt_v, [idxs], ones16)
        return carry

    lax.fori_loop(0, EPW // 16, hist_body, 0)
    n_tail = EPW - (EPW // 16) * 16
    if n_tail:
        idxs = idxf_v[pl.ds(EPW - 16, 16)]
        tail_mask = lax.iota(jnp.int32, 16) >= (16 - n_tail)
        plsc.addupdate_scatter(cnt_v, [idxs], ones16, mask=tail_mask)

    def scat_drain(j, carry):
        pltpu.make_async_copy(tp_v.at[pl.ds(j * CHUNK, CHUNK)],
                              acc_sh.at[idx_v.at[j]], sem).wait()
        return carry

    lax.fori_loop(0, NCH, scat_drain, 0)
    plsc.subcore_barrier()

    pltpu.sync_copy(acc_sh.at[pl.ds(sid * ROWS_PER_TILE, ROWS_PER_TILE)],
                    psum_hbm.at[cid].at[pl.ds(sid * ROWS_PER_TILE, ROWS_PER_TILE)])
    pltpu.sync_copy(cnt_v, pcnt_hbm.at[wid])


BLK = 1000
NBLK = Q // BLK
KC = HID * C


def _edge_body(x_ref, *refs):
    ea_refs = refs[0:8]
    sht_ref = refs[8]
    w1_ref, b1_ref, m3_ref, w2r_ref, rs8_ref, eye_ref, tp_ref = refs[9:]
    w1 = w1_ref[...]
    b1 = b1_ref[...]
    w2r = w2r_ref[...]
    xf = x_ref[...]
    m3 = m3_ref[...]
    eye = eye_ref[...]
    dn0 = (((0,), (1,)), ((), ()))
    dnt = (((0,), (0,)), ((), ()))
    ea_all = jnp.concatenate([r[...] for r in ea_refs], axis=0)
    ht_all = jnp.maximum(
        lax.dot_general(w1, ea_all, dn0,
                        preferred_element_type=jnp.float32) + b1, 0.0)
    xt_all = jnp.concatenate(
        [lax.dot_general(eye, xf[:, 16 * j:16 * (j + 1)], dn0,
                         preferred_element_type=jnp.float32)
         for j in range(8)], axis=1)
    gt = jnp.concatenate([ht_all, xt_all], axis=0)
    yt = lax.dot_general(m3, gt, dnt,
                         preferred_element_type=jnp.float32)
    zt = yt[:KC] * yt[KC:2 * KC]
    tpt = lax.dot_general(w2r, zt, dnt,
                          preferred_element_type=jnp.float32)
    tpt = tpt + yt[2 * KC:]
    tp_all = jnp.transpose(jnp.concatenate(
        [tpt[:, j * BLK:(j + 1) * BLK] for j in range(8)], axis=0))
    sh8t = sht_ref[0]
    sh_all = jnp.dot(sh8t.T, rs8_ref[...],
                     preferred_element_type=jnp.float32)
    tp_ref[...] = tp_all * (sh_all * ALPHA)


_edge_tc = pl.pallas_call(
    _edge_body,
    out_shape=jax.ShapeDtypeStruct((Q, 128), jnp.float32),
    grid=(NBLK,),
    in_specs=(
        [pl.BlockSpec((BLK, 128), lambda i: (i, 0))]
        + [pl.BlockSpec((BLK, NEF), lambda i, j=j: (j * NBLK + i, 0))
           for j in range(8)]
        + [pl.BlockSpec((1, 8, BLK), lambda i: (i, 0, 0))]
        + [
            pl.BlockSpec((NEF, HID), lambda i: (0, 0)),
            pl.BlockSpec((HID, 1), lambda i: (0, 0)),
            pl.BlockSpec((2 * HID, 2 * KC + D), lambda i: (0, 0)),
            pl.BlockSpec((KC, D), lambda i: (0, 0)),
            pl.BlockSpec((8, 128), lambda i: (0, 0)),
            pl.BlockSpec((C, C), lambda i: (0, 0)),
        ]
    ),
    out_specs=pl.BlockSpec((BLK, 128), lambda i: (i, 0)),
    compiler_params=pltpu.CompilerParams(
        dimension_semantics=("parallel",)),
)

_RS8 = np.kron(np.eye(8, dtype=np.float32), np.ones((1, 16), np.float32))

_RH = np.kron(np.eye(HID, dtype=np.float32), np.ones((1, C), np.float32))
_RX = np.tile(np.eye(C, dtype=np.float32), (1, HID))


def _fin_body(psum_ref, pcnt_ref, na_ref, bnw_ref, bnb_ref, out_ref):
    s = psum_ref[0] + psum_ref[1]
    cnt = jnp.sum(pcnt_ref[...], axis=0)
    o = s / jnp.maximum(cnt, 1.0)[:, None] + na_ref[...]
    m = jnp.mean(o, axis=0, keepdims=True)
    v = jnp.mean((o - m) ** 2, axis=0, keepdims=True)
    out_ref[...] = (o - m) * lax.rsqrt(v + 1e-5) * bnw_ref[...] + bnb_ref[...]


_finalize_tc = pl.pallas_call(
    _fin_body,
    out_shape=jax.ShapeDtypeStruct((N, D), jnp.float32),
)


@functools.lru_cache(maxsize=1)
def _build_sc_kernels():
    mesh = plsc.VectorSubcoreMesh(core_axis_name="c", subcore_axis_name="s")
    sc_params = pltpu.CompilerParams(use_tc_tiling_on_sc=False,
                                     needs_layout_passes=False)
    gather = pl.kernel(
        _gather_rows_body,
        out_type=jax.ShapeDtypeStruct((Q, 128), jnp.float32),
        mesh=mesh,
        compiler_params=sc_params,
        scratch_types=[
            pltpu.VMEM((NCH, CHUNK), jnp.int32),
            pltpu.VMEM((EPW, C), jnp.float32),
            pltpu.SemaphoreType.DMA,
        ],
    )
    scatter = pl.kernel(
        _scatter_mean_parts_body,
        out_type=(
            jax.ShapeDtypeStruct((NC, N, D), jnp.float32),
            jax.ShapeDtypeStruct((NW, N), jnp.float32),
        ),
        mesh=mesh,
        compiler_params=sc_params,
        scratch_types=[
            pltpu.VMEM((NCH, CHUNK), jnp.int32),
            pltpu.VMEM((EPW,), jnp.int32),
            pltpu.VMEM((EPW, D), jnp.float32),
            pltpu.VMEM((N,), jnp.float32),
            pltpu.VMEM((ROWS_PER_TILE, D), jnp.float32),
            pltpu.VMEM_SHARED((N, D), jnp.float32),
            pltpu.SemaphoreType.DMA,
        ],
    )
    return gather, scatter


def kernel(node_attr, edge_index, edge_attr, edge_sh, global_graph_embedding,
           ptr, W1, b1, W2, b2, bn_w, bn_b):
    _gather_rows, _scatter_mean_parts = _build_sc_kernels()
    edge_src = edge_index[0]
    edge_dst = edge_index[1]
    dst2d = edge_dst.reshape(NW * NCH, CHUNK)
    src2d = edge_src.reshape(NW * NCH, CHUNK)

    x = _gather_rows(node_attr, dst2d)

    w2r = W2.reshape(HID, C, D).reshape(HID * C, D)
    bm = b2.reshape(C, D)
    m3 = jnp.concatenate([
        jnp.concatenate([_RH, jnp.zeros((HID, KC + D), jnp.float32)], axis=1),
        jnp.concatenate([jnp.zeros((C, KC), jnp.float32), _RX, bm], axis=1),
    ], axis=0)
    sht = jnp.transpose(edge_sh.reshape(8, NBLK, BLK), (1, 0, 2))
    tp128 = _edge_tc(x, *([edge_attr] * 8), sht, W1,
                     b1.reshape(HID, 1), m3, w2r,
                     jnp.asarray(_RS8), jnp.eye(C, dtype=jnp.float32))

    psum, pcnt = _scatter_mean_parts(tp128, src2d, edge_src)

    return _finalize_tc(psum, pcnt, node_attr, bn_w.reshape(1, D),
                        bn_b.reshape(1, D))
